# baseline (speedup 1.0000x reference)
import jax
import jax.numpy as jnp
from jax.experimental import pallas as pl

N, D = 100000, 64
BR = 4000
G = N // BR


def _body(conc_ref, w_ref, b_ref, tab_ref, out_ref):
    out_ref[...] = tab_ref[...] + conc_ref[0] * w_ref[...] + b_ref[...]


def kernel(initial_concentration, W_conc, b_conc, identity_table):
    conc3 = initial_concentration.reshape(G, BR, 1)
    w = W_conc.reshape(1, D)
    b = b_conc.reshape(1, D)
    out = pl.pallas_call(
        _body,
        grid=(G,),
        in_specs=[
            pl.BlockSpec((1, BR, 1), lambda i: (i, 0, 0)),
            pl.BlockSpec((1, D), lambda i: (0, 0)),
            pl.BlockSpec((1, D), lambda i: (0, 0)),
            pl.BlockSpec((BR, D), lambda i: (i, 0)),
        ],
        out_specs=pl.BlockSpec((BR, D), lambda i: (i, 0)),
        out_shape=jax.ShapeDtypeStruct((N, D), jnp.float32),
    )(conc3, w, b, identity_table)
    return out


# speedup vs baseline: 1.2020x; 1.2020x over previous
import functools

import jax
import jax.numpy as jnp
from jax import lax
from jax.experimental import pallas as pl
from jax.experimental.pallas import tpu as pltpu
from jax.experimental.pallas import tpu_sc as plsc

N, D = 100000, 64
L = 16
NC, NS = 2, 16
NW = NC * NS
R = 400
C = N // R
CPW = (C + NW - 1) // NW


def _sc_body(conc_hbm, w_hbm, b_hbm, table_hbm, out_hbm, buf, concbuf, wbuf, bbuf):
    wid = lax.axis_index("s") * NC + lax.axis_index("c")
    pltpu.sync_copy(w_hbm, wbuf)
    pltpu.sync_copy(b_hbm, bbuf)
    wv = [wbuf[pl.ds(L * j, L)] for j in range(D // L)]
    bv = [bbuf[pl.ds(L * j, L)] for j in range(D // L)]

    def chunk_body(ci, _):
        cid = ci * NW + wid

        @pl.when(cid < C)
        def _():
            base = cid * R
            pltpu.sync_copy(table_hbm.at[pl.ds(base, R)], buf)
            pltpu.sync_copy(conc_hbm.at[pl.ds(base, R)], concbuf)

            def grp_body(g, _):
                c16 = concbuf[pl.ds(L * g, L)]
                for k in range(L):
                    cv = jnp.full((L,), c16[k])
                    r = L * g + k
                    for j in range(D // L):
                        sl = pl.ds(L * j, L)
                        buf[r, sl] = buf[r, sl] + (cv * wv[j] + bv[j])
                return 0

            lax.fori_loop(0, R // L, grp_body, 0)
            pltpu.sync_copy(buf, out_hbm.at[pl.ds(base, R)])

        return 0

    lax.fori_loop(0, CPW, chunk_body, 0)


def kernel(initial_concentration, W_conc, b_conc, identity_table):
    mesh = plsc.VectorSubcoreMesh(core_axis_name="c", subcore_axis_name="s")
    f = functools.partial(
        pl.kernel,
        mesh=mesh,
        out_type=jax.ShapeDtypeStruct((N, D), jnp.float32),
        scratch_types=[
            pltpu.VMEM((R, D), jnp.float32),
            pltpu.VMEM((R,), jnp.float32),
            pltpu.VMEM((D,), jnp.float32),
            pltpu.VMEM((D,), jnp.float32),
        ],
    )(_sc_body)
    return f(initial_concentration, W_conc.reshape(D), b_conc, identity_table)


# speedup vs baseline: 1.3414x; 1.1160x over previous
import functools

import jax
import jax.numpy as jnp
from jax import lax
from jax.experimental import pallas as pl
from jax.experimental.pallas import tpu as pltpu
from jax.experimental.pallas import tpu_sc as plsc

N, D = 100000, 64
L = 16
NC, NS = 2, 16
NW = NC * NS
R = 400
C = N // R
CPW = (C + NW - 1) // NW


def _sc_body(conc_hbm, w_hbm, b_hbm, table_hbm, out_hbm,
             buf0, buf1, cb0, cb1, wbuf, bbuf,
             lsem0, lsem1, csem0, csem1, ssem0, ssem1):
    wid = lax.axis_index("s") * NC + lax.axis_index("c")
    pltpu.sync_copy(w_hbm, wbuf)
    pltpu.sync_copy(b_hbm, bbuf)
    wv = [wbuf[pl.ds(L * j, L)] for j in range(D // L)]
    bv = [bbuf[pl.ds(L * j, L)] for j in range(D // L)]

    bufs = [buf0, buf1]
    cbs = [cb0, cb1]
    lsems = [lsem0, lsem1]
    csems = [csem0, csem1]
    ssems = [ssem0, ssem1]

    def load(i):
        p = i % 2
        cid = i * NW + wid

        @pl.when(cid < C)
        def _():
            base = cid * R
            pltpu.make_async_copy(table_hbm.at[pl.ds(base, R)], bufs[p], lsems[p]).start()
            pltpu.make_async_copy(conc_hbm.at[pl.ds(base, R)], cbs[p], csems[p]).start()

    def compute_store(i):
        p = i % 2
        cid = i * NW + wid

        @pl.when(cid < C)
        def _():
            base = cid * R
            buf, cb = bufs[p], cbs[p]
            pltpu.make_async_copy(table_hbm.at[pl.ds(base, R)], buf, lsems[p]).wait()
            pltpu.make_async_copy(conc_hbm.at[pl.ds(base, R)], cb, csems[p]).wait()

            def grp_body(g, _):
                c16 = cb[pl.ds(L * g, L)]
                for k in range(L):
                    cv = jnp.full((L,), c16[k])
                    r = L * g + k
                    for j in range(D // L):
                        sl = pl.ds(L * j, L)
                        buf[r, sl] = buf[r, sl] + (cv * wv[j] + bv[j])
                return 0

            lax.fori_loop(0, R // L, grp_body, 0)
            pltpu.make_async_copy(buf, out_hbm.at[pl.ds(base, R)], ssems[p]).start()

    def wait_store(i):
        p = i % 2
        cid = i * NW + wid

        @pl.when(cid < C)
        def _():
            base = cid * R
            pltpu.make_async_copy(bufs[p], out_hbm.at[pl.ds(base, R)], ssems[p]).wait()

    load(0)
    for i in range(CPW):
        if i + 1 < CPW:
            if i >= 1:
                wait_store(i - 1)
            load(i + 1)
        compute_store(i)
    if CPW >= 2:
        wait_store(CPW - 2)
    wait_store(CPW - 1)


def kernel(initial_concentration, W_conc, b_conc, identity_table):
    mesh = plsc.VectorSubcoreMesh(core_axis_name="c", subcore_axis_name="s")
    f = functools.partial(
        pl.kernel,
        mesh=mesh,
        out_type=jax.ShapeDtypeStruct((N, D), jnp.float32),
        scratch_types=[
            pltpu.VMEM((R, D), jnp.float32),
            pltpu.VMEM((R, D), jnp.float32),
            pltpu.VMEM((R,), jnp.float32),
            pltpu.VMEM((R,), jnp.float32),
            pltpu.VMEM((D,), jnp.float32),
            pltpu.VMEM((D,), jnp.float32),
            pltpu.SemaphoreType.DMA,
            pltpu.SemaphoreType.DMA,
            pltpu.SemaphoreType.DMA,
            pltpu.SemaphoreType.DMA,
            pltpu.SemaphoreType.DMA,
            pltpu.SemaphoreType.DMA,
        ],
    )(_sc_body)
    return f(initial_concentration, W_conc.reshape(D), b_conc, identity_table)


# speedup vs baseline: 1.5150x; 1.1294x over previous
import jax
import jax.numpy as jnp
from jax import lax
from jax.experimental import pallas as pl

N, D = 100000, 64
BR = 4096
G = (N + BR - 1) // BR


def _body(conc_ref, w_ref, b_ref, tab_ref, out_ref):
    cm = conc_ref[...].reshape(1, BR)
    outer = lax.dot_general(cm, w_ref[...], (((0,), (0,)), ((), ())),
                            preferred_element_type=jnp.float32)
    out_ref[...] = tab_ref[...] + outer + b_ref[...]


def kernel(initial_concentration, W_conc, b_conc, identity_table):
    w = W_conc.reshape(1, D)
    b = b_conc.reshape(1, D)
    out = pl.pallas_call(
        _body,
        grid=(G,),
        in_specs=[
            pl.BlockSpec((BR,), lambda i: (i,)),
            pl.BlockSpec((1, D), lambda i: (0, 0)),
            pl.BlockSpec((1, D), lambda i: (0, 0)),
            pl.BlockSpec((BR, D), lambda i: (i, 0)),
        ],
        out_specs=pl.BlockSpec((BR, D), lambda i: (i, 0)),
        out_shape=jax.ShapeDtypeStruct((N, D), jnp.float32),
    )(initial_concentration, w, b, identity_table)
    return out
